# baseline (speedup 1.0000x reference)
import functools

import jax
import jax.numpy as jnp
from jax import lax
from jax.experimental import pallas as pl
from jax.experimental.pallas import tpu as pltpu
from jax.experimental.pallas import tpu_sc as plsc

NC = 2
NS = 16
L = 16
NW = NC * NS
CB = 128
G = 8


def _sc_degree(dst_r, nr, gc0, gc1):
    chunks = dst_r.shape[1]
    per_tile = nr // NS
    mesh = plsc.VectorSubcoreMesh(core_axis_name="c", subcore_axis_name="s")

    @functools.partial(
        pl.kernel, mesh=mesh,
        out_type=jax.ShapeDtypeStruct((NC, nr), jnp.float32),
        scratch_types=[
            pltpu.VMEM((chunks, CB), jnp.int32),
            pltpu.VMEM((CB,), jnp.float32),
            pltpu.VMEM((per_tile,), jnp.float32),
            pltpu.VMEM_SHARED((nr,), jnp.float32),
        ],
    )
    def k(dst_hbm, deg_hbm, idx_v, ones_v, zbuf_v, acc_sh):
        cid = lax.axis_index("c")
        sid = lax.axis_index("s")
        wid = cid * NS + sid

        @pl.loop(0, CB // L)
        def _(i):
            ones_v[pl.ds(i * L, L)] = jnp.ones((L,), jnp.float32)

        @pl.loop(0, per_tile // L)
        def _(i):
            zbuf_v[pl.ds(i * L, L)] = jnp.zeros((L,), jnp.float32)

        pltpu.sync_copy(zbuf_v, acc_sh.at[pl.ds(sid * per_tile, per_tile)])
        plsc.subcore_barrier()

        pltpu.sync_copy(dst_hbm.at[wid], idx_v)
        chunks_c = jnp.where(cid == 0, gc0, gc1) * G

        @pl.loop(0, chunks_c)
        def _(j):
            pltpu.sync_copy(ones_v, acc_sh.at[idx_v.at[j]], add=True)

        plsc.subcore_barrier()
        pltpu.sync_copy(acc_sh.at[pl.ds(sid * per_tile, per_tile)],
                        deg_hbm.at[cid, pl.ds(sid * per_tile, per_tile)])

    return k(dst_r)


def _sc_scatter(hp, src_r, dst_r, nr, gc0, gc1):
    chunks = src_r.shape[1]
    d = hp.shape[1]
    rows_per_tile = nr // NS
    zrows = 64
    copies = rows_per_tile // zrows
    assert chunks % G == 0 and G % 2 == 0 and min(gc0, gc1) >= 2
    mesh = plsc.VectorSubcoreMesh(core_axis_name="c", subcore_axis_name="s")

    @functools.partial(
        pl.kernel, mesh=mesh,
        out_type=jax.ShapeDtypeStruct((NC, nr, d), jnp.float32),
        scratch_types=[
            pltpu.VMEM((2, G, CB), jnp.int32),
            pltpu.VMEM((2, G, CB), jnp.int32),
            pltpu.VMEM((2, CB, d), jnp.float32),
            pltpu.SemaphoreType.DMA((2,)),
            pltpu.SemaphoreType.DMA((2,)),
            pltpu.VMEM_SHARED((nr, d), jnp.float32),
        ],
    )
    def k(hp_hbm, src_hbm, dst_hbm, out_hbm,
          sidx_v, didx_v, rows_v, gsem, isem, acc_sh):
        cid = lax.axis_index("c")
        sid = lax.axis_index("s")
        wid = cid * NS + sid

        @pl.loop(0, zrows)
        def _(r):
            for c in range(d // L):
                rows_v[0, r, pl.ds(c * L, L)] = jnp.zeros((L,), jnp.float32)

        for kc in range(copies):
            pltpu.sync_copy(
                rows_v.at[0, pl.ds(0, zrows)],
                acc_sh.at[pl.ds((sid * copies + kc) * zrows, zrows)])
        plsc.subcore_barrier()

        pltpu.sync_copy(src_hbm.at[wid, pl.ds(0, G)], sidx_v.at[0])
        pltpu.sync_copy(dst_hbm.at[wid, pl.ds(0, G)], didx_v.at[0])
        for b in range(2):
            pltpu.async_copy(hp_hbm.at[sidx_v.at[0, b]], rows_v.at[b], gsem.at[b])
        pltpu.async_copy(src_hbm.at[wid, pl.ds(G, G)], sidx_v.at[1], isem.at[1])
        pltpu.async_copy(dst_hbm.at[wid, pl.ds(G, G)], didx_v.at[1], isem.at[1])

        groups_c = jnp.where(cid == 0, gc0, gc1)

        @pl.loop(0, groups_c)
        def _(g):
            gb = lax.rem(g, 2)
            nb = lax.rem(g + 1, 2)
            not_last = g < groups_c - 1

            @pl.when(jnp.logical_and(g >= 1, not_last))
            def _():
                pltpu.async_copy(src_hbm.at[wid, pl.ds((g + 1) * G, G)],
                                 sidx_v.at[nb], isem.at[nb])
                pltpu.async_copy(dst_hbm.at[wid, pl.ds((g + 1) * G, G)],
                                 didx_v.at[nb], isem.at[nb])

            for jp in range(G):
                b = jp % 2
                pltpu.make_async_copy(hp_hbm.at[sidx_v.at[gb, jp]],
                                      rows_v.at[b], gsem.at[b]).wait()
                pltpu.sync_copy(rows_v.at[b], acc_sh.at[didx_v.at[gb, jp]],
                                add=True)
                if jp < G - 2:
                    pltpu.async_copy(hp_hbm.at[sidx_v.at[gb, jp + 2]],
                                     rows_v.at[b], gsem.at[b])
                else:
                    if jp == G - 2:
                        @pl.when(not_last)
                        def _():
                            pltpu.make_async_copy(
                                src_hbm.at[wid, pl.ds(0, G)],
                                sidx_v.at[nb], isem.at[nb]).wait()
                            pltpu.make_async_copy(
                                dst_hbm.at[wid, pl.ds(0, G)],
                                didx_v.at[nb], isem.at[nb]).wait()

                    @pl.when(not_last)
                    def _():
                        pltpu.async_copy(hp_hbm.at[sidx_v.at[nb, jp + 2 - G]],
                                         rows_v.at[b], gsem.at[b])

        plsc.subcore_barrier()
        pltpu.sync_copy(acc_sh.at[pl.ds(sid * rows_per_tile, rows_per_tile)],
                        out_hbm.at[cid, pl.ds(sid * rows_per_tile, rows_per_tile)])

    return k(hp, src_r, dst_r)


def _tc_matmul_scale(x, W, deg0, deg1, blk):
    n, d_in = x.shape
    d_out = W.shape[1]
    g = n // blk

    def body(x_ref, w_ref, d0_ref, d1_ref, hp_ref, dinv_ref):
        h = jnp.dot(x_ref[...], w_ref[...], preferred_element_type=jnp.float32)
        deg = d0_ref[0, 0, :] + d1_ref[0, 0, :] + 1.0
        dinv = lax.rsqrt(deg)
        hp_ref[...] = h * dinv[:, None]
        dinv_ref[0, 0, :] = dinv

    return pl.pallas_call(
        body,
        grid=(g,),
        in_specs=[
            pl.BlockSpec((blk, d_in), lambda i: (i, 0)),
            pl.BlockSpec((d_in, d_out), lambda i: (0, 0)),
            pl.BlockSpec((1, 1, blk), lambda i: (i, 0, 0)),
            pl.BlockSpec((1, 1, blk), lambda i: (i, 0, 0)),
        ],
        out_specs=[
            pl.BlockSpec((blk, d_out), lambda i: (i, 0)),
            pl.BlockSpec((1, 1, blk), lambda i: (i, 0, 0)),
        ],
        out_shape=[
            jax.ShapeDtypeStruct((n, d_out), jnp.float32),
            jax.ShapeDtypeStruct((g, 1, blk), jnp.float32),
        ],
    )(x, W, deg0, deg1)


def _tc_combine(P, hp, dinv3, b2, a2, blk):
    n, d_out = hp.shape
    g = n // blk

    def body(p_ref, hp_ref, dinv_ref, b_ref, a_ref, o_ref):
        s = p_ref[0] + p_ref[1] + hp_ref[...]
        dinv = dinv_ref[0, 0, :]
        o = s * dinv[:, None] + b_ref[0, :][None, :]
        o_ref[...] = jnp.where(o >= 0, o, a_ref[0, :][None, :] * o)

    return pl.pallas_call(
        body,
        grid=(g,),
        in_specs=[
            pl.BlockSpec((NC, blk, d_out), lambda i: (0, i, 0)),
            pl.BlockSpec((blk, d_out), lambda i: (i, 0)),
            pl.BlockSpec((1, 1, blk), lambda i: (i, 0, 0)),
            pl.BlockSpec((1, d_out), lambda i: (0, 0)),
            pl.BlockSpec((1, d_out), lambda i: (0, 0)),
        ],
        out_specs=pl.BlockSpec((blk, d_out), lambda i: (i, 0)),
        out_shape=jax.ShapeDtypeStruct((n, d_out), jnp.float32),
    )(P, hp, dinv3, b2, a2)


def _lay(arr, padval, gc, gmax):
    cap = NS * gc * G * CB
    a = jnp.concatenate([arr, jnp.full((cap - arr.shape[0],), padval, jnp.int32)])
    a = a.reshape(NS, gc * G * CB)
    a = jnp.pad(a, ((0, 0), (0, (gmax - gc) * G * CB)), constant_values=padval)
    return a.reshape(NS, gmax * G, CB)


def kernel(x, edge_index, W, b, prelu_a):
    n, d_in = x.shape
    d_out = W.shape[1]
    e = edge_index.shape[1]

    gtot = -(-e // (NS * G * CB))
    gc0 = max(2, round(gtot * 0.25))
    gc1 = gtot - gc0
    gmax = max(gc0, gc1)
    chunks = gmax * G
    nr = 2048 * (-(-(n + 1) // 2048))

    src = edge_index[0].astype(jnp.int32)
    dst = edge_index[1].astype(jnp.int32)
    ec0 = NS * gc0 * G * CB
    srcp = jnp.concatenate([_lay(src[:ec0], 0, gc0, gmax),
                            _lay(src[ec0:], 0, gc1, gmax)], axis=0)
    dstp = jnp.concatenate([_lay(dst[:ec0], n, gc0, gmax),
                            _lay(dst[ec0:], n, gc1, gmax)], axis=0)

    degp = _sc_degree(dstp, nr, gc0, gc1)

    blk = 200
    assert n % blk == 0
    g = n // blk
    deg0 = degp[0, :n].reshape(g, 1, blk)
    deg1 = degp[1, :n].reshape(g, 1, blk)

    hp, dinv3 = _tc_matmul_scale(x, W, deg0, deg1, blk)
    P = _sc_scatter(hp, srcp, dstp, nr, gc0, gc1)
    out = _tc_combine(P, hp, dinv3,
                      b.reshape(1, d_out), prelu_a.reshape(1, d_out), blk)
    return out


# speedup vs baseline: 1.0676x; 1.0676x over previous
import functools

import jax
import jax.numpy as jnp
from jax import lax
from jax.experimental import pallas as pl
from jax.experimental.pallas import tpu as pltpu
from jax.experimental.pallas import tpu_sc as plsc

NC = 2
NS = 16
L = 16
NW = NC * NS
CB = 128
G = 8


def _sc_degree(dst_r, nr, gc0, gc1):
    chunks = dst_r.shape[1]
    per_tile = nr // NS
    mesh = plsc.VectorSubcoreMesh(core_axis_name="c", subcore_axis_name="s")

    @functools.partial(
        pl.kernel, mesh=mesh,
        out_type=jax.ShapeDtypeStruct((NC, nr), jnp.float32),
        scratch_types=[
            pltpu.VMEM((chunks, CB), jnp.int32),
            pltpu.VMEM((CB,), jnp.float32),
            pltpu.VMEM((per_tile,), jnp.float32),
            pltpu.VMEM_SHARED((nr,), jnp.float32),
        ],
    )
    def k(dst_hbm, deg_hbm, idx_v, ones_v, zbuf_v, acc_sh):
        cid = lax.axis_index("c")
        sid = lax.axis_index("s")
        wid = cid * NS + sid

        @pl.loop(0, CB // L)
        def _(i):
            ones_v[pl.ds(i * L, L)] = jnp.ones((L,), jnp.float32)

        @pl.loop(0, per_tile // L)
        def _(i):
            zbuf_v[pl.ds(i * L, L)] = jnp.zeros((L,), jnp.float32)

        pltpu.sync_copy(zbuf_v, acc_sh.at[pl.ds(sid * per_tile, per_tile)])
        plsc.subcore_barrier()

        pltpu.sync_copy(dst_hbm.at[wid], idx_v)
        chunks_c = jnp.where(cid == 0, gc0, gc1) * G

        @pl.loop(0, chunks_c)
        def _(j):
            pltpu.sync_copy(ones_v, acc_sh.at[idx_v.at[j]], add=True)

        plsc.subcore_barrier()
        pltpu.sync_copy(acc_sh.at[pl.ds(sid * per_tile, per_tile)],
                        deg_hbm.at[cid, pl.ds(sid * per_tile, per_tile)])

    return k(dst_r)


def _sc_scatter(hp, src_r, dst_r, nr, gc0, gc1):
    chunks = src_r.shape[1]
    d = hp.shape[1]
    rows_per_tile = nr // NS
    zrows = 64
    copies = rows_per_tile // zrows
    assert chunks % G == 0 and G % 2 == 0 and min(gc0, gc1) >= 2
    mesh = plsc.VectorSubcoreMesh(core_axis_name="c", subcore_axis_name="s")

    @functools.partial(
        pl.kernel, mesh=mesh,
        out_type=jax.ShapeDtypeStruct((NC, nr, d), jnp.float32),
        scratch_types=[
            pltpu.VMEM((2, G, CB), jnp.int32),
            pltpu.VMEM((2, G, CB), jnp.int32),
            pltpu.VMEM((2, CB, d), jnp.float32),
            pltpu.SemaphoreType.DMA((2,)),
            pltpu.SemaphoreType.DMA((2,)),
            pltpu.VMEM_SHARED((nr, d), jnp.float32),
        ],
    )
    def k(hp_hbm, src_hbm, dst_hbm, out_hbm,
          sidx_v, didx_v, rows_v, gsem, isem, acc_sh):
        cid = lax.axis_index("c")
        sid = lax.axis_index("s")
        wid = cid * NS + sid

        @pl.loop(0, zrows)
        def _(r):
            for c in range(d // L):
                rows_v[0, r, pl.ds(c * L, L)] = jnp.zeros((L,), jnp.float32)

        for kc in range(copies):
            pltpu.sync_copy(
                rows_v.at[0, pl.ds(0, zrows)],
                acc_sh.at[pl.ds((sid * copies + kc) * zrows, zrows)])
        plsc.subcore_barrier()

        pltpu.sync_copy(src_hbm.at[wid, pl.ds(0, G)], sidx_v.at[0])
        pltpu.sync_copy(dst_hbm.at[wid, pl.ds(0, G)], didx_v.at[0])
        for b in range(2):
            pltpu.async_copy(hp_hbm.at[sidx_v.at[0, b]], rows_v.at[b], gsem.at[b])
        pltpu.async_copy(src_hbm.at[wid, pl.ds(G, G)], sidx_v.at[1], isem.at[1])
        pltpu.async_copy(dst_hbm.at[wid, pl.ds(G, G)], didx_v.at[1], isem.at[1])

        groups_c = jnp.where(cid == 0, gc0, gc1)

        @pl.loop(0, groups_c)
        def _(g):
            gb = lax.rem(g, 2)
            nb = lax.rem(g + 1, 2)
            not_last = g < groups_c - 1

            @pl.when(jnp.logical_and(g >= 1, not_last))
            def _():
                pltpu.async_copy(src_hbm.at[wid, pl.ds((g + 1) * G, G)],
                                 sidx_v.at[nb], isem.at[nb])
                pltpu.async_copy(dst_hbm.at[wid, pl.ds((g + 1) * G, G)],
                                 didx_v.at[nb], isem.at[nb])

            for jp in range(G):
                b = jp % 2
                pltpu.make_async_copy(hp_hbm.at[sidx_v.at[gb, jp]],
                                      rows_v.at[b], gsem.at[b]).wait()
                pltpu.sync_copy(rows_v.at[b], acc_sh.at[didx_v.at[gb, jp]],
                                add=True)
                if jp < G - 2:
                    pltpu.async_copy(hp_hbm.at[sidx_v.at[gb, jp + 2]],
                                     rows_v.at[b], gsem.at[b])
                else:
                    if jp == G - 2:
                        @pl.when(not_last)
                        def _():
                            pltpu.make_async_copy(
                                src_hbm.at[wid, pl.ds(0, G)],
                                sidx_v.at[nb], isem.at[nb]).wait()
                            pltpu.make_async_copy(
                                dst_hbm.at[wid, pl.ds(0, G)],
                                didx_v.at[nb], isem.at[nb]).wait()

                    @pl.when(not_last)
                    def _():
                        pltpu.async_copy(hp_hbm.at[sidx_v.at[nb, jp + 2 - G]],
                                         rows_v.at[b], gsem.at[b])

        plsc.subcore_barrier()
        pltpu.sync_copy(acc_sh.at[pl.ds(sid * rows_per_tile, rows_per_tile)],
                        out_hbm.at[cid, pl.ds(sid * rows_per_tile, rows_per_tile)])

    return k(hp, src_r, dst_r)


def _tc_matmul_scale(x, W, deg0, deg1, blk):
    n, d_in = x.shape
    d_out = W.shape[1]
    g = n // blk

    def body(x_ref, w_ref, d0_ref, d1_ref, hp_ref, dinv_ref):
        h = jnp.dot(x_ref[...], w_ref[...], preferred_element_type=jnp.float32)
        deg = d0_ref[0, 0, :] + d1_ref[0, 0, :] + 1.0
        dinv = lax.rsqrt(deg)
        hp_ref[...] = h * dinv[:, None]
        dinv_ref[0, 0, :] = dinv

    return pl.pallas_call(
        body,
        grid=(g,),
        in_specs=[
            pl.BlockSpec((blk, d_in), lambda i: (i, 0)),
            pl.BlockSpec((d_in, d_out), lambda i: (0, 0)),
            pl.BlockSpec((1, 1, blk), lambda i: (i, 0, 0)),
            pl.BlockSpec((1, 1, blk), lambda i: (i, 0, 0)),
        ],
        out_specs=[
            pl.BlockSpec((blk, d_out), lambda i: (i, 0)),
            pl.BlockSpec((1, 1, blk), lambda i: (i, 0, 0)),
        ],
        out_shape=[
            jax.ShapeDtypeStruct((n, d_out), jnp.float32),
            jax.ShapeDtypeStruct((g, 1, blk), jnp.float32),
        ],
    )(x, W, deg0, deg1)


def _tc_combine(P, hp, dinv3, b2, a2, blk):
    n, d_out = hp.shape
    g = n // blk

    def body(p_ref, hp_ref, dinv_ref, b_ref, a_ref, o_ref):
        s = p_ref[0] + p_ref[1] + hp_ref[...]
        dinv = dinv_ref[0, 0, :]
        o = s * dinv[:, None] + b_ref[0, :][None, :]
        o_ref[...] = jnp.where(o >= 0, o, a_ref[0, :][None, :] * o)

    return pl.pallas_call(
        body,
        grid=(g,),
        in_specs=[
            pl.BlockSpec((NC, blk, d_out), lambda i: (0, i, 0)),
            pl.BlockSpec((blk, d_out), lambda i: (i, 0)),
            pl.BlockSpec((1, 1, blk), lambda i: (i, 0, 0)),
            pl.BlockSpec((1, d_out), lambda i: (0, 0)),
            pl.BlockSpec((1, d_out), lambda i: (0, 0)),
        ],
        out_specs=pl.BlockSpec((blk, d_out), lambda i: (i, 0)),
        out_shape=jax.ShapeDtypeStruct((n, d_out), jnp.float32),
    )(P, hp, dinv3, b2, a2)


def _lay(arr, padval, gc, gmax):
    cap = NS * gc * G * CB
    a = jnp.concatenate([arr, jnp.full((cap - arr.shape[0],), padval, jnp.int32)])
    a = a.reshape(NS, gc * G * CB)
    a = jnp.pad(a, ((0, 0), (0, (gmax - gc) * G * CB)), constant_values=padval)
    return a.reshape(NS, gmax * G, CB)


def kernel(x, edge_index, W, b, prelu_a):
    n, d_in = x.shape
    d_out = W.shape[1]
    e = edge_index.shape[1]

    gtot = -(-e // (NS * G * CB))
    gc1 = max(2, round(gtot * 0.25))
    gc0 = gtot - gc1
    gmax = max(gc0, gc1)
    chunks = gmax * G
    nr = 2048 * (-(-(n + 1) // 2048))

    src = edge_index[0].astype(jnp.int32)
    dst = edge_index[1].astype(jnp.int32)
    ec0 = NS * gc0 * G * CB
    srcp = jnp.concatenate([_lay(src[:ec0], 0, gc0, gmax),
                            _lay(src[ec0:], 0, gc1, gmax)], axis=0)
    dstp = jnp.concatenate([_lay(dst[:ec0], n, gc0, gmax),
                            _lay(dst[ec0:], n, gc1, gmax)], axis=0)

    degp = _sc_degree(dstp, nr, gc0, gc1)

    blk = 200
    assert n % blk == 0
    g = n // blk
    deg0 = degp[0, :n].reshape(g, 1, blk)
    deg1 = degp[1, :n].reshape(g, 1, blk)

    hp, dinv3 = _tc_matmul_scale(x, W, deg0, deg1, blk)
    P = _sc_scatter(hp, srcp, dstp, nr, gc0, gc1)
    out = _tc_combine(P, hp, dinv3,
                      b.reshape(1, d_out), prelu_a.reshape(1, d_out), blk)
    return out


# speedup vs baseline: 2.5391x; 2.3783x over previous
import functools

import jax
import jax.numpy as jnp
from jax import lax
from jax.experimental import pallas as pl
from jax.experimental.pallas import tpu as pltpu
from jax.experimental.pallas import tpu_sc as plsc

NC = 2
NS = 16
L = 16
NW = NC * NS
CB = 128
G = 8


def _sc_degree(dst_r, nr, gc0, gc1):
    chunks = dst_r.shape[1]
    per_tile = nr // NS
    mesh = plsc.VectorSubcoreMesh(core_axis_name="c", subcore_axis_name="s")

    @functools.partial(
        pl.kernel, mesh=mesh,
        out_type=jax.ShapeDtypeStruct((NC, nr), jnp.float32),
        scratch_types=[
            pltpu.VMEM((chunks, CB), jnp.int32),
            pltpu.VMEM((CB,), jnp.float32),
            pltpu.VMEM((per_tile,), jnp.float32),
            pltpu.VMEM_SHARED((nr,), jnp.float32),
        ],
    )
    def k(dst_hbm, deg_hbm, idx_v, ones_v, zbuf_v, acc_sh):
        cid = lax.axis_index("c")
        sid = lax.axis_index("s")
        wid = cid * NS + sid

        @pl.loop(0, CB // L)
        def _(i):
            ones_v[pl.ds(i * L, L)] = jnp.ones((L,), jnp.float32)

        @pl.loop(0, per_tile // L)
        def _(i):
            zbuf_v[pl.ds(i * L, L)] = jnp.zeros((L,), jnp.float32)

        pltpu.sync_copy(zbuf_v, acc_sh.at[pl.ds(sid * per_tile, per_tile)])
        plsc.subcore_barrier()

        pltpu.sync_copy(dst_hbm.at[wid], idx_v)
        chunks_c = jnp.where(cid == 0, gc0, gc1) * G

        @pl.loop(0, chunks_c)
        def _(j):
            pltpu.sync_copy(ones_v, acc_sh.at[idx_v.at[j]], add=True)

        plsc.subcore_barrier()
        pltpu.sync_copy(acc_sh.at[pl.ds(sid * per_tile, per_tile)],
                        deg_hbm.at[cid, pl.ds(sid * per_tile, per_tile)])

    return k(dst_r)


def _sc_scatter(hp, src_r, dst_r, nr, gc0, gc1):
    chunks = src_r.shape[1]
    d = hp.shape[1]
    rows_per_tile = nr // NS
    zrows = 64
    copies = rows_per_tile // zrows
    assert chunks % G == 0 and G % 2 == 0 and min(gc0, gc1) >= 2
    mesh = plsc.VectorSubcoreMesh(core_axis_name="c", subcore_axis_name="s")

    @functools.partial(
        pl.kernel, mesh=mesh,
        out_type=jax.ShapeDtypeStruct((NC, nr, d), jnp.float32),
        scratch_types=[
            pltpu.VMEM((2, G, CB), jnp.int32),
            pltpu.VMEM((2, G, CB), jnp.int32),
            pltpu.VMEM((2, CB, d), jnp.float32),
            pltpu.SemaphoreType.DMA((2,)),
            pltpu.SemaphoreType.DMA((2,)),
            pltpu.VMEM_SHARED((nr, d), jnp.float32),
        ],
    )
    def k(hp_hbm, src_hbm, dst_hbm, out_hbm,
          sidx_v, didx_v, rows_v, gsem, isem, acc_sh):
        cid = lax.axis_index("c")
        sid = lax.axis_index("s")
        wid = cid * NS + sid

        @pl.loop(0, zrows)
        def _(r):
            for c in range(d // L):
                rows_v[0, r, pl.ds(c * L, L)] = jnp.zeros((L,), jnp.float32)

        for kc in range(copies):
            pltpu.sync_copy(
                rows_v.at[0, pl.ds(0, zrows)],
                acc_sh.at[pl.ds((sid * copies + kc) * zrows, zrows)])
        plsc.subcore_barrier()

        pltpu.sync_copy(src_hbm.at[wid, pl.ds(0, G)], sidx_v.at[0])
        pltpu.sync_copy(dst_hbm.at[wid, pl.ds(0, G)], didx_v.at[0])
        for b in range(2):
            pltpu.async_copy(hp_hbm.at[sidx_v.at[0, b]], rows_v.at[b], gsem.at[b])
        pltpu.async_copy(src_hbm.at[wid, pl.ds(G, G)], sidx_v.at[1], isem.at[1])
        pltpu.async_copy(dst_hbm.at[wid, pl.ds(G, G)], didx_v.at[1], isem.at[1])

        groups_c = jnp.where(cid == 0, gc0, gc1)

        @pl.loop(0, groups_c)
        def _(g):
            gb = lax.rem(g, 2)
            nb = lax.rem(g + 1, 2)
            not_last = g < groups_c - 1

            @pl.when(jnp.logical_and(g >= 1, not_last))
            def _():
                pltpu.async_copy(src_hbm.at[wid, pl.ds((g + 1) * G, G)],
                                 sidx_v.at[nb], isem.at[nb])
                pltpu.async_copy(dst_hbm.at[wid, pl.ds((g + 1) * G, G)],
                                 didx_v.at[nb], isem.at[nb])

            for jp in range(G):
                b = jp % 2
                pltpu.make_async_copy(hp_hbm.at[sidx_v.at[gb, jp]],
                                      rows_v.at[b], gsem.at[b]).wait()
                pltpu.sync_copy(rows_v.at[b], acc_sh.at[didx_v.at[gb, jp]],
                                add=True)
                if jp < G - 2:
                    pltpu.async_copy(hp_hbm.at[sidx_v.at[gb, jp + 2]],
                                     rows_v.at[b], gsem.at[b])
                else:
                    if jp == G - 2:
                        @pl.when(not_last)
                        def _():
                            pltpu.make_async_copy(
                                src_hbm.at[wid, pl.ds(0, G)],
                                sidx_v.at[nb], isem.at[nb]).wait()
                            pltpu.make_async_copy(
                                dst_hbm.at[wid, pl.ds(0, G)],
                                didx_v.at[nb], isem.at[nb]).wait()

                    @pl.when(not_last)
                    def _():
                        pltpu.async_copy(hp_hbm.at[sidx_v.at[nb, jp + 2 - G]],
                                         rows_v.at[b], gsem.at[b])

        plsc.subcore_barrier()
        pltpu.sync_copy(acc_sh.at[pl.ds(sid * rows_per_tile, rows_per_tile)],
                        out_hbm.at[cid, pl.ds(sid * rows_per_tile, rows_per_tile)])

    return k(hp, src_r, dst_r)


def _tc_matmul_scale(x, W, deg0, deg1, blk):
    n, d_in = x.shape
    d_out = W.shape[1]
    g = n // blk

    def body(x_ref, w_ref, d0_ref, d1_ref, hp_ref, dinv_ref):
        h = jnp.dot(x_ref[...], w_ref[...], preferred_element_type=jnp.float32)
        deg = d0_ref[0, 0, :] + d1_ref[0, 0, :] + 1.0
        dinv = lax.rsqrt(deg)
        hp_ref[...] = h * dinv[:, None]
        dinv_ref[0, 0, :] = dinv

    return pl.pallas_call(
        body,
        grid=(g,),
        in_specs=[
            pl.BlockSpec((blk, d_in), lambda i: (i, 0)),
            pl.BlockSpec((d_in, d_out), lambda i: (0, 0)),
            pl.BlockSpec((1, 1, blk), lambda i: (i, 0, 0)),
            pl.BlockSpec((1, 1, blk), lambda i: (i, 0, 0)),
        ],
        out_specs=[
            pl.BlockSpec((blk, d_out), lambda i: (i, 0)),
            pl.BlockSpec((1, 1, blk), lambda i: (i, 0, 0)),
        ],
        out_shape=[
            jax.ShapeDtypeStruct((n, d_out), jnp.float32),
            jax.ShapeDtypeStruct((g, 1, blk), jnp.float32),
        ],
    )(x, W, deg0, deg1)


def _tc_combine(P, hp, dinv3, b2, a2, blk):
    n, d_out = hp.shape
    g = n // blk

    def body(p_ref, hp_ref, dinv_ref, b_ref, a_ref, o_ref):
        s = p_ref[0] + p_ref[1] + hp_ref[...]
        dinv = dinv_ref[0, 0, :]
        o = s * dinv[:, None] + b_ref[0, :][None, :]
        o_ref[...] = jnp.where(o >= 0, o, a_ref[0, :][None, :] * o)

    return pl.pallas_call(
        body,
        grid=(g,),
        in_specs=[
            pl.BlockSpec((NC, blk, d_out), lambda i: (0, i, 0)),
            pl.BlockSpec((blk, d_out), lambda i: (i, 0)),
            pl.BlockSpec((1, 1, blk), lambda i: (i, 0, 0)),
            pl.BlockSpec((1, d_out), lambda i: (0, 0)),
            pl.BlockSpec((1, d_out), lambda i: (0, 0)),
        ],
        out_specs=pl.BlockSpec((blk, d_out), lambda i: (i, 0)),
        out_shape=jax.ShapeDtypeStruct((n, d_out), jnp.float32),
    )(P, hp, dinv3, b2, a2)


def _lay(arr, padval, gc, gmax):
    cap = NS * gc * G * CB
    a = jnp.concatenate([arr, jnp.full((cap - arr.shape[0],), padval, jnp.int32)])
    a = a.reshape(NS, gc * G * CB)
    a = jnp.pad(a, ((0, 0), (0, (gmax - gc) * G * CB)), constant_values=padval)
    return a.reshape(NS, gmax * G, CB)


def kernel(x, edge_index, W, b, prelu_a):
    n, d_in = x.shape
    d_out = W.shape[1]
    e = edge_index.shape[1]

    gtot = 2 * (-(-e // (2 * NS * G * CB)))
    gc0 = gtot // 2
    gc1 = gtot - gc0
    gmax = max(gc0, gc1)
    chunks = gmax * G
    nr = 2048 * (-(-(n + 1) // 2048))

    src = edge_index[0].astype(jnp.int32)
    dst = edge_index[1].astype(jnp.int32)
    ep = NS * gtot * G * CB
    pad = ep - e
    pad_src = jnp.arange(pad, dtype=jnp.int32) % n
    pad_dst = n + (jnp.arange(pad, dtype=jnp.int32) % (nr - n))
    src = jnp.concatenate([src, pad_src])
    dst = jnp.concatenate([dst, pad_dst])
    ec0 = NS * gc0 * G * CB
    srcp = jnp.concatenate([_lay(src[:ec0], 0, gc0, gmax),
                            _lay(src[ec0:], 0, gc1, gmax)], axis=0)
    dstp = jnp.concatenate([_lay(dst[:ec0], n, gc0, gmax),
                            _lay(dst[ec0:], n, gc1, gmax)], axis=0)

    degp = _sc_degree(dstp, nr, gc0, gc1)

    blk = 200
    assert n % blk == 0
    g = n // blk
    deg0 = degp[0, :n].reshape(g, 1, blk)
    deg1 = degp[1, :n].reshape(g, 1, blk)

    hp, dinv3 = _tc_matmul_scale(x, W, deg0, deg1, blk)
    P = _sc_scatter(hp, srcp, dstp, nr, gc0, gc1)
    out = _tc_combine(P, hp, dinv3,
                      b.reshape(1, d_out), prelu_a.reshape(1, d_out), blk)
    return out


# speedup vs baseline: 3.1414x; 1.2372x over previous
import functools

import jax
import jax.numpy as jnp
from jax import lax
from jax.experimental import pallas as pl
from jax.experimental.pallas import tpu as pltpu
from jax.experimental.pallas import tpu_sc as plsc

NC = 2
NS = 16
L = 16
NW = NC * NS
CB = 128
G = 8


def _sc_degree(dst_r, nr, gc0, gc1):
    chunks = dst_r.shape[1]
    per_tile = nr // NS
    mesh = plsc.VectorSubcoreMesh(core_axis_name="c", subcore_axis_name="s")

    @functools.partial(
        pl.kernel, mesh=mesh,
        out_type=jax.ShapeDtypeStruct((NC, nr), jnp.float32),
        scratch_types=[
            pltpu.VMEM((chunks, CB), jnp.int32),
            pltpu.VMEM((CB,), jnp.float32),
            pltpu.VMEM((per_tile,), jnp.float32),
            pltpu.VMEM_SHARED((nr,), jnp.float32),
        ],
    )
    def k(dst_hbm, deg_hbm, idx_v, ones_v, zbuf_v, acc_sh):
        cid = lax.axis_index("c")
        sid = lax.axis_index("s")
        wid = cid * NS + sid

        @pl.loop(0, CB // L)
        def _(i):
            ones_v[pl.ds(i * L, L)] = jnp.ones((L,), jnp.float32)

        @pl.loop(0, per_tile // L)
        def _(i):
            zbuf_v[pl.ds(i * L, L)] = jnp.zeros((L,), jnp.float32)

        pltpu.sync_copy(zbuf_v, acc_sh.at[pl.ds(sid * per_tile, per_tile)])
        plsc.subcore_barrier()

        pltpu.sync_copy(dst_hbm.at[wid], idx_v)
        chunks_c = jnp.where(cid == 0, gc0, gc1) * G

        @pl.loop(0, chunks_c)
        def _(j):
            pltpu.sync_copy(ones_v, acc_sh.at[idx_v.at[j]], add=True)

        plsc.subcore_barrier()
        pltpu.sync_copy(acc_sh.at[pl.ds(sid * per_tile, per_tile)],
                        deg_hbm.at[cid, pl.ds(sid * per_tile, per_tile)])

    return k(dst_r)


def _sc_scatter(hp, src_r, dst_r, nr, gc0, gc1):
    chunks = src_r.shape[1]
    d = hp.shape[1]
    rows_per_tile = nr // NS
    zrows = 64
    copies = rows_per_tile // zrows
    assert chunks % G == 0 and G % 2 == 0 and min(gc0, gc1) >= 2
    mesh = plsc.VectorSubcoreMesh(core_axis_name="c", subcore_axis_name="s")

    @functools.partial(
        pl.kernel, mesh=mesh,
        out_type=jax.ShapeDtypeStruct((NC, nr, d), jnp.float32),
        scratch_types=[
            pltpu.VMEM((2, G, CB), jnp.int32),
            pltpu.VMEM((2, G, CB), jnp.int32),
            pltpu.VMEM((2, CB, d), jnp.float32),
            pltpu.SemaphoreType.DMA((2,)),
            pltpu.SemaphoreType.DMA((2,)),
            pltpu.VMEM_SHARED((nr, d), jnp.float32),
        ],
    )
    def k(hp_hbm, src_hbm, dst_hbm, out_hbm,
          sidx_v, didx_v, rows_v, gsem, isem, acc_sh):
        cid = lax.axis_index("c")
        sid = lax.axis_index("s")
        wid = cid * NS + sid

        @pl.loop(0, zrows)
        def _(r):
            for c in range(d // L):
                rows_v[0, r, pl.ds(c * L, L)] = jnp.zeros((L,), jnp.float32)

        for kc in range(copies):
            pltpu.sync_copy(
                rows_v.at[0, pl.ds(0, zrows)],
                acc_sh.at[pl.ds((sid * copies + kc) * zrows, zrows)])
        plsc.subcore_barrier()

        pltpu.sync_copy(src_hbm.at[wid, pl.ds(0, G)], sidx_v.at[0])
        pltpu.sync_copy(dst_hbm.at[wid, pl.ds(0, G)], didx_v.at[0])
        for b in range(2):
            pltpu.async_copy(hp_hbm.at[sidx_v.at[0, b]], rows_v.at[b], gsem.at[b])
        pltpu.async_copy(src_hbm.at[wid, pl.ds(G, G)], sidx_v.at[1], isem.at[1])
        pltpu.async_copy(dst_hbm.at[wid, pl.ds(G, G)], didx_v.at[1], isem.at[1])

        groups_c = jnp.where(cid == 0, gc0, gc1)

        @pl.loop(0, groups_c)
        def _(g):
            gb = lax.rem(g, 2)
            nb = lax.rem(g + 1, 2)
            not_last = g < groups_c - 1

            @pl.when(jnp.logical_and(g >= 1, not_last))
            def _():
                pltpu.async_copy(src_hbm.at[wid, pl.ds((g + 1) * G, G)],
                                 sidx_v.at[nb], isem.at[nb])
                pltpu.async_copy(dst_hbm.at[wid, pl.ds((g + 1) * G, G)],
                                 didx_v.at[nb], isem.at[nb])

            for jp in range(G):
                b = jp % 2
                pltpu.make_async_copy(hp_hbm.at[sidx_v.at[gb, jp]],
                                      rows_v.at[b], gsem.at[b]).wait()
                pltpu.sync_copy(rows_v.at[b], acc_sh.at[didx_v.at[gb, jp]],
                                add=True)
                if jp < G - 2:
                    pltpu.async_copy(hp_hbm.at[sidx_v.at[gb, jp + 2]],
                                     rows_v.at[b], gsem.at[b])
                else:
                    if jp == G - 2:
                        @pl.when(not_last)
                        def _():
                            pltpu.make_async_copy(
                                src_hbm.at[wid, pl.ds(0, G)],
                                sidx_v.at[nb], isem.at[nb]).wait()
                            pltpu.make_async_copy(
                                dst_hbm.at[wid, pl.ds(0, G)],
                                didx_v.at[nb], isem.at[nb]).wait()

                    @pl.when(not_last)
                    def _():
                        pltpu.async_copy(hp_hbm.at[sidx_v.at[nb, jp + 2 - G]],
                                         rows_v.at[b], gsem.at[b])

        plsc.subcore_barrier()
        pltpu.sync_copy(acc_sh.at[pl.ds(sid * rows_per_tile, rows_per_tile)],
                        out_hbm.at[cid, pl.ds(sid * rows_per_tile, rows_per_tile)])

    return k(hp, src_r, dst_r)


def _tc_matmul_scale(x, W, deg0, deg1, blk):
    n, d_in = x.shape
    d_out = W.shape[1]
    g = n // blk

    def body(x_ref, w_ref, d0_ref, d1_ref, hp_ref, dinv_ref):
        h = jnp.dot(x_ref[...], w_ref[...], preferred_element_type=jnp.float32)
        deg = d0_ref[0, 0, :] + d1_ref[0, 0, :] + 1.0
        dinv = lax.rsqrt(deg)
        hp_ref[...] = h * dinv[:, None]
        dinv_ref[0, 0, :] = dinv

    return pl.pallas_call(
        body,
        grid=(g,),
        in_specs=[
            pl.BlockSpec((blk, d_in), lambda i: (i, 0)),
            pl.BlockSpec((d_in, d_out), lambda i: (0, 0)),
            pl.BlockSpec((1, 1, blk), lambda i: (i, 0, 0)),
            pl.BlockSpec((1, 1, blk), lambda i: (i, 0, 0)),
        ],
        out_specs=[
            pl.BlockSpec((blk, d_out), lambda i: (i, 0)),
            pl.BlockSpec((1, 1, blk), lambda i: (i, 0, 0)),
        ],
        out_shape=[
            jax.ShapeDtypeStruct((n, d_out), jnp.float32),
            jax.ShapeDtypeStruct((g, 1, blk), jnp.float32),
        ],
    )(x, W, deg0, deg1)


def _tc_combine(P, hp, dinv3, b2, a2, blk):
    n, d_out = hp.shape
    g = n // blk

    def body(p_ref, hp_ref, dinv_ref, b_ref, a_ref, o_ref):
        s = p_ref[0] + p_ref[1] + hp_ref[...]
        dinv = dinv_ref[0, 0, :]
        o = s * dinv[:, None] + b_ref[0, :][None, :]
        o_ref[...] = jnp.where(o >= 0, o, a_ref[0, :][None, :] * o)

    return pl.pallas_call(
        body,
        grid=(g,),
        in_specs=[
            pl.BlockSpec((NC, blk, d_out), lambda i: (0, i, 0)),
            pl.BlockSpec((blk, d_out), lambda i: (i, 0)),
            pl.BlockSpec((1, 1, blk), lambda i: (i, 0, 0)),
            pl.BlockSpec((1, d_out), lambda i: (0, 0)),
            pl.BlockSpec((1, d_out), lambda i: (0, 0)),
        ],
        out_specs=pl.BlockSpec((blk, d_out), lambda i: (i, 0)),
        out_shape=jax.ShapeDtypeStruct((n, d_out), jnp.float32),
    )(P, hp, dinv3, b2, a2)


def _lay(arr, padval, gc, gmax):
    cap = NS * gc * G * CB
    a = jnp.concatenate([arr, jnp.full((cap - arr.shape[0],), padval, jnp.int32)])
    a = a.reshape(NS, gc * G * CB)
    a = jnp.pad(a, ((0, 0), (0, (gmax - gc) * G * CB)), constant_values=padval)
    return a.reshape(NS, gmax * G, CB)


def kernel(x, edge_index, W, b, prelu_a):
    n, d_in = x.shape
    d_out = W.shape[1]
    e = edge_index.shape[1]

    gtot = 2 * (-(-e // (2 * NS * G * CB)))
    gc0 = gtot // 2
    gc1 = gtot - gc0
    gmax = max(gc0, gc1)
    chunks = gmax * G
    nr = 2048 * (-(-(n + 1) // 2048))

    src = edge_index[0].astype(jnp.int32)
    dst = edge_index[1].astype(jnp.int32)
    ep = NS * gtot * G * CB
    pad = ep - e
    pad_src = jnp.arange(pad, dtype=jnp.int32) % n
    pad_dst = n + (jnp.arange(pad, dtype=jnp.int32) % (nr - n))
    src = jnp.concatenate([src, pad_src])
    dst = jnp.concatenate([dst, pad_dst])
    ec0 = NS * gc0 * G * CB
    srcp = jnp.concatenate([_lay(src[:ec0], 0, gc0, gmax),
                            _lay(src[ec0:], 0, gc1, gmax)], axis=0)
    dstp = jnp.concatenate([_lay(dst[:ec0], n, gc0, gmax),
                            _lay(dst[ec0:], n, gc1, gmax)], axis=0)

    degp = _sc_degree(dstp, nr, gc0, gc1)

    blk = 1000
    assert n % blk == 0
    g = n // blk
    deg0 = degp[0, :n].reshape(g, 1, blk)
    deg1 = degp[1, :n].reshape(g, 1, blk)

    hp, dinv3 = _tc_matmul_scale(x, W, deg0, deg1, blk)
    P = _sc_scatter(hp, srcp, dstp, nr, gc0, gc1)
    out = _tc_combine(P, hp, dinv3,
                      b.reshape(1, d_out), prelu_a.reshape(1, d_out), blk)
    return out


# speedup vs baseline: 3.1476x; 1.0019x over previous
import functools

import jax
import jax.numpy as jnp
import numpy as np
from jax import lax
from jax.experimental import pallas as pl
from jax.experimental.pallas import tpu as pltpu
from jax.experimental.pallas import tpu_sc as plsc

NC = 2
NS = 16
L = 16
NW = NC * NS
CB = 128
G = 8


def _sc_degree(dst_r, nr, gc0, gc1):
    chunks = dst_r.shape[1]
    per_tile = nr // NS
    mesh = plsc.VectorSubcoreMesh(core_axis_name="c", subcore_axis_name="s")

    @functools.partial(
        pl.kernel, mesh=mesh,
        out_type=jax.ShapeDtypeStruct((NC, nr), jnp.float32),
        scratch_types=[
            pltpu.VMEM((chunks, CB), jnp.int32),
            pltpu.VMEM((CB,), jnp.float32),
            pltpu.VMEM((per_tile,), jnp.float32),
            pltpu.VMEM_SHARED((nr,), jnp.float32),
        ],
    )
    def k(dst_hbm, deg_hbm, idx_v, ones_v, zbuf_v, acc_sh):
        cid = lax.axis_index("c")
        sid = lax.axis_index("s")
        wid = cid * NS + sid

        @pl.loop(0, CB // L)
        def _(i):
            ones_v[pl.ds(i * L, L)] = jnp.ones((L,), jnp.float32)

        @pl.loop(0, per_tile // L)
        def _(i):
            zbuf_v[pl.ds(i * L, L)] = jnp.zeros((L,), jnp.float32)

        pltpu.sync_copy(zbuf_v, acc_sh.at[pl.ds(sid * per_tile, per_tile)])
        plsc.subcore_barrier()

        pltpu.sync_copy(dst_hbm.at[wid], idx_v)
        chunks_c = jnp.where(cid == 0, gc0, gc1) * G

        @pl.loop(0, chunks_c)
        def _(j):
            pltpu.sync_copy(ones_v, acc_sh.at[idx_v.at[j]], add=True)

        plsc.subcore_barrier()
        pltpu.sync_copy(acc_sh.at[pl.ds(sid * per_tile, per_tile)],
                        deg_hbm.at[cid, pl.ds(sid * per_tile, per_tile)])

    return k(dst_r)


def _sc_scatter(hp, src_r, dst_r, nr, gc0, gc1):
    chunks = src_r.shape[1]
    d = hp.shape[1]
    rows_per_tile = nr // NS
    zrows = 64
    copies = rows_per_tile // zrows
    assert chunks % G == 0 and G % 2 == 0 and min(gc0, gc1) >= 2
    mesh = plsc.VectorSubcoreMesh(core_axis_name="c", subcore_axis_name="s")

    @functools.partial(
        pl.kernel, mesh=mesh,
        out_type=jax.ShapeDtypeStruct((NC, nr, d), jnp.float32),
        scratch_types=[
            pltpu.VMEM((2, G, CB), jnp.int32),
            pltpu.VMEM((2, G, CB), jnp.int32),
            pltpu.VMEM((2, CB, d), jnp.float32),
            pltpu.SemaphoreType.DMA((2,)),
            pltpu.SemaphoreType.DMA((2,)),
            pltpu.VMEM_SHARED((nr, d), jnp.float32),
        ],
    )
    def k(hp_hbm, src_hbm, dst_hbm, out_hbm,
          sidx_v, didx_v, rows_v, gsem, isem, acc_sh):
        cid = lax.axis_index("c")
        sid = lax.axis_index("s")
        wid = cid * NS + sid

        @pl.loop(0, zrows)
        def _(r):
            for c in range(d // L):
                rows_v[0, r, pl.ds(c * L, L)] = jnp.zeros((L,), jnp.float32)

        for kc in range(copies):
            pltpu.sync_copy(
                rows_v.at[0, pl.ds(0, zrows)],
                acc_sh.at[pl.ds((sid * copies + kc) * zrows, zrows)])
        plsc.subcore_barrier()

        pltpu.sync_copy(src_hbm.at[wid, pl.ds(0, G)], sidx_v.at[0])
        pltpu.sync_copy(dst_hbm.at[wid, pl.ds(0, G)], didx_v.at[0])
        for b in range(2):
            pltpu.async_copy(hp_hbm.at[sidx_v.at[0, b]], rows_v.at[b], gsem.at[b])
        pltpu.async_copy(src_hbm.at[wid, pl.ds(G, G)], sidx_v.at[1], isem.at[1])
        pltpu.async_copy(dst_hbm.at[wid, pl.ds(G, G)], didx_v.at[1], isem.at[1])

        groups_c = jnp.where(cid == 0, gc0, gc1)

        @pl.loop(0, groups_c)
        def _(g):
            gb = lax.rem(g, 2)
            nb = lax.rem(g + 1, 2)
            not_last = g < groups_c - 1

            @pl.when(jnp.logical_and(g >= 1, not_last))
            def _():
                pltpu.async_copy(src_hbm.at[wid, pl.ds((g + 1) * G, G)],
                                 sidx_v.at[nb], isem.at[nb])
                pltpu.async_copy(dst_hbm.at[wid, pl.ds((g + 1) * G, G)],
                                 didx_v.at[nb], isem.at[nb])

            for jp in range(G):
                b = jp % 2
                pltpu.make_async_copy(hp_hbm.at[sidx_v.at[gb, jp]],
                                      rows_v.at[b], gsem.at[b]).wait()
                pltpu.sync_copy(rows_v.at[b], acc_sh.at[didx_v.at[gb, jp]],
                                add=True)
                if jp < G - 2:
                    pltpu.async_copy(hp_hbm.at[sidx_v.at[gb, jp + 2]],
                                     rows_v.at[b], gsem.at[b])
                else:
                    if jp == G - 2:
                        @pl.when(not_last)
                        def _():
                            pltpu.make_async_copy(
                                src_hbm.at[wid, pl.ds(0, G)],
                                sidx_v.at[nb], isem.at[nb]).wait()
                            pltpu.make_async_copy(
                                dst_hbm.at[wid, pl.ds(0, G)],
                                didx_v.at[nb], isem.at[nb]).wait()

                    @pl.when(not_last)
                    def _():
                        pltpu.async_copy(hp_hbm.at[sidx_v.at[nb, jp + 2 - G]],
                                         rows_v.at[b], gsem.at[b])

        plsc.subcore_barrier()
        pltpu.sync_copy(acc_sh.at[pl.ds(sid * rows_per_tile, rows_per_tile)],
                        out_hbm.at[cid, pl.ds(sid * rows_per_tile, rows_per_tile)])

    return k(hp, src_r, dst_r)


def _tc_matmul(x, W, blk):
    n, d_in = x.shape
    d_out = W.shape[1]
    g = n // blk

    def body(x_ref, w_ref, h_ref):
        h_ref[...] = jnp.dot(x_ref[...], w_ref[...],
                             preferred_element_type=jnp.float32)

    return pl.pallas_call(
        body,
        grid=(g,),
        in_specs=[
            pl.BlockSpec((blk, d_in), lambda i: (i, 0)),
            pl.BlockSpec((d_in, d_out), lambda i: (0, 0)),
        ],
        out_specs=pl.BlockSpec((blk, d_out), lambda i: (i, 0)),
        out_shape=jax.ShapeDtypeStruct((n, d_out), jnp.float32),
    )(x, W)


def _tc_scale(h, deg0, deg1, blk):
    n, d_out = h.shape
    g = n // blk

    def body(h_ref, d0_ref, d1_ref, hp_ref, dinv_ref):
        deg = d0_ref[0, 0, :] + d1_ref[0, 0, :] + 1.0
        dinv = lax.rsqrt(deg)
        hp_ref[...] = h_ref[...] * dinv[:, None]
        dinv_ref[0, 0, :] = dinv

    return pl.pallas_call(
        body,
        grid=(g,),
        in_specs=[
            pl.BlockSpec((blk, d_out), lambda i: (i, 0)),
            pl.BlockSpec((1, 1, blk), lambda i: (i, 0, 0)),
            pl.BlockSpec((1, 1, blk), lambda i: (i, 0, 0)),
        ],
        out_specs=[
            pl.BlockSpec((blk, d_out), lambda i: (i, 0)),
            pl.BlockSpec((1, 1, blk), lambda i: (i, 0, 0)),
        ],
        out_shape=[
            jax.ShapeDtypeStruct((n, d_out), jnp.float32),
            jax.ShapeDtypeStruct((g, 1, blk), jnp.float32),
        ],
    )(h, deg0, deg1)


def _tc_combine(P, hp, dinv3, b2, a2, blk):
    n, d_out = hp.shape
    g = n // blk

    def body(p_ref, hp_ref, dinv_ref, b_ref, a_ref, o_ref):
        s = p_ref[0] + p_ref[1] + hp_ref[...]
        dinv = dinv_ref[0, 0, :]
        o = s * dinv[:, None] + b_ref[0, :][None, :]
        o_ref[...] = jnp.where(o >= 0, o, a_ref[0, :][None, :] * o)

    return pl.pallas_call(
        body,
        grid=(g,),
        in_specs=[
            pl.BlockSpec((NC, blk, d_out), lambda i: (0, i, 0)),
            pl.BlockSpec((blk, d_out), lambda i: (i, 0)),
            pl.BlockSpec((1, 1, blk), lambda i: (i, 0, 0)),
            pl.BlockSpec((1, d_out), lambda i: (0, 0)),
            pl.BlockSpec((1, d_out), lambda i: (0, 0)),
        ],
        out_specs=pl.BlockSpec((blk, d_out), lambda i: (i, 0)),
        out_shape=jax.ShapeDtypeStruct((n, d_out), jnp.float32),
    )(P, hp, dinv3, b2, a2)


def _lay(arr, padval, gc, gmax):
    cap = NS * gc * G * CB
    a = jnp.concatenate([arr, jnp.full((cap - arr.shape[0],), padval, jnp.int32)])
    a = a.reshape(NS, gc * G * CB)
    a = jnp.pad(a, ((0, 0), (0, (gmax - gc) * G * CB)), constant_values=padval)
    return a.reshape(NS, gmax * G, CB)


def kernel(x, edge_index, W, b, prelu_a):
    n, d_in = x.shape
    d_out = W.shape[1]
    e = edge_index.shape[1]

    gtot = 2 * (-(-e // (2 * NS * G * CB)))
    gc0 = gtot // 2
    gc1 = gtot - gc0
    gmax = max(gc0, gc1)
    chunks = gmax * G
    nr = 2048 * (-(-(n + 1) // 2048))

    src = edge_index[0].astype(jnp.int32)
    dst = edge_index[1].astype(jnp.int32)
    ep = NS * gtot * G * CB
    pad = ep - e
    pad_src = jnp.asarray(np.arange(pad) % n, dtype=jnp.int32)
    pad_dst = jnp.asarray(n + (np.arange(pad) % (nr - n)), dtype=jnp.int32)
    src = jnp.concatenate([src, pad_src])
    dst = jnp.concatenate([dst, pad_dst])
    ec0 = NS * gc0 * G * CB
    srcp = jnp.concatenate([_lay(src[:ec0], 0, gc0, gmax),
                            _lay(src[ec0:], 0, gc1, gmax)], axis=0)
    dstp = jnp.concatenate([_lay(dst[:ec0], n, gc0, gmax),
                            _lay(dst[ec0:], n, gc1, gmax)], axis=0)

    degp = _sc_degree(dstp, nr, gc0, gc1)

    blk = 1000
    assert n % blk == 0
    g = n // blk
    deg0 = degp[0, :n].reshape(g, 1, blk)
    deg1 = degp[1, :n].reshape(g, 1, blk)

    h = _tc_matmul(x, W, blk)
    hp, dinv3 = _tc_scale(h, deg0, deg1, blk)
    P = _sc_scatter(hp, srcp, dstp, nr, gc0, gc1)
    out = _tc_combine(P, hp, dinv3,
                      b.reshape(1, d_out), prelu_a.reshape(1, d_out), blk)
    return out
